# baseline (speedup 1.0000x reference)
import functools

import jax
import jax.numpy as jnp
from jax import lax
from jax.experimental import pallas as pl
from jax.experimental.pallas import tpu as pltpu
from jax.experimental.pallas import tpu_sc as plsc


def _topk_idx_body(scores_ref, idxp_ref, idxe_ref):
    N = scores_ref.shape[-1]
    K = idxp_ref.shape[-1]
    b = pl.program_id(0)
    s = scores_ref[0]
    scol = jnp.reshape(s, (N, 1))
    ni = lax.broadcasted_iota(jnp.int32, (N, N), 0)
    mi = lax.broadcasted_iota(jnp.int32, (N, N), 1)
    beats = (s > scol) | ((s == scol) & (mi < ni))
    rank = jnp.sum(beats.astype(jnp.int32), axis=1, keepdims=True)
    jrow = lax.broadcasted_iota(jnp.int32, (1, K), 1)
    sel = rank == jrow
    nidx = lax.broadcasted_iota(jnp.int32, (N, K), 0)
    idx = jnp.sum(jnp.where(sel, nidx, 0), axis=0, keepdims=True)
    idxp_ref[0] = idx + b * N
    idxe_ref[0] = idx + 1


def _topk_indices(scores, B, N, K):
    scores3 = scores.reshape(B, 1, N)
    idxp, idxe = pl.pallas_call(
        _topk_idx_body,
        grid=(B,),
        in_specs=[pl.BlockSpec((1, 1, N), lambda b: (b, 0, 0))],
        out_specs=[
            pl.BlockSpec((1, 1, K), lambda b: (b, 0, 0)),
            pl.BlockSpec((1, 1, K), lambda b: (b, 0, 0)),
        ],
        out_shape=[
            jax.ShapeDtypeStruct((B, 1, K), jnp.int32),
            jax.ShapeDtypeStruct((B, 1, K), jnp.int32),
        ],
    )(scores3)
    return idxp.reshape(B * K), idxe.reshape(B * K)


_NC, _NS = 2, 16
_NW = _NC * _NS
_CHUNK = 64


def _sc_gather_body(magno_hbm, pos_hbm, idxp_hbm, idxe_hbm, out_hbm,
                    idxp_v, idxe_v, rows_v, pose_v, sem1, sem2):
    D = rows_v.shape[-1]
    rows_total = out_hbm.shape[0]
    rows_per_w = rows_total // _NW
    nchunk = rows_per_w // _CHUNK
    wid = lax.axis_index("s") * _NC + lax.axis_index("c")
    base = wid * rows_per_w

    def chunk(ci, carry):
        off = pl.multiple_of(base + ci * _CHUNK, _CHUNK)
        pltpu.sync_copy(idxp_hbm.at[pl.ds(off, _CHUNK)], idxp_v)
        pltpu.sync_copy(idxe_hbm.at[pl.ds(off, _CHUNK)], idxe_v)
        c1 = pltpu.async_copy(magno_hbm.at[idxp_v], rows_v, sem1)
        c2 = pltpu.async_copy(pos_hbm.at[idxe_v], pose_v, sem2)
        c1.wait()
        c2.wait()

        def addrow(r, c):
            for d0 in range(0, D, 16):
                rows_v[r, pl.ds(d0, 16)] = (
                    rows_v[r, pl.ds(d0, 16)] + pose_v[r, pl.ds(d0, 16)])
            return c

        lax.fori_loop(0, _CHUNK, addrow, 0)
        pltpu.sync_copy(rows_v, out_hbm.at[pl.ds(off, _CHUNK)])
        return carry

    lax.fori_loop(0, nchunk, chunk, 0)


def _sc_gather(magno_flat, pos_flat, idxp, idxe, rows, D):
    mesh = plsc.VectorSubcoreMesh(core_axis_name="c", subcore_axis_name="s")
    return pl.kernel(
        _sc_gather_body,
        out_type=jax.ShapeDtypeStruct((rows, D), jnp.float32),
        mesh=mesh,
        scratch_types=[
            pltpu.VMEM((_CHUNK,), jnp.int32),
            pltpu.VMEM((_CHUNK,), jnp.int32),
            pltpu.VMEM((_CHUNK, D), jnp.float32),
            pltpu.VMEM((_CHUNK, D), jnp.float32),
            pltpu.SemaphoreType.DMA,
            pltpu.SemaphoreType.DMA,
        ],
    )(magno_flat, pos_flat, idxp, idxe)


def kernel(magno_patches, vit_positional_embedding, scores):
    B, N, D = magno_patches.shape
    K = N // 4
    idxp, idxe = _topk_indices(scores, B, N, K)
    magno_flat = magno_patches.reshape(B * N, D)
    pos_flat = vit_positional_embedding[0]
    out = _sc_gather(magno_flat, pos_flat, idxp, idxe, B * K, D)
    return out.reshape(B, K, D)


# speedup vs baseline: 1.2343x; 1.2343x over previous
import functools

import jax
import jax.numpy as jnp
from jax import lax
from jax.experimental import pallas as pl
from jax.experimental.pallas import tpu as pltpu
from jax.experimental.pallas import tpu_sc as plsc


def _topk_idx_body(scores_ref, idxp_ref, idxe_ref):
    N = scores_ref.shape[-1]
    K = idxp_ref.shape[-1]
    b = pl.program_id(0)
    s = scores_ref[0]
    scol = jnp.reshape(s, (N, 1))
    ni = lax.broadcasted_iota(jnp.int32, (N, N), 0)
    mi = lax.broadcasted_iota(jnp.int32, (N, N), 1)
    beats = (s > scol) | ((s == scol) & (mi < ni))
    rank = jnp.sum(beats.astype(jnp.int32), axis=1, keepdims=True)
    jrow = lax.broadcasted_iota(jnp.int32, (1, K), 1)
    sel = rank == jrow
    nidx = lax.broadcasted_iota(jnp.int32, (N, K), 0)
    idx = jnp.sum(jnp.where(sel, nidx, 0), axis=0, keepdims=True)
    idxp_ref[0] = idx + b * N
    idxe_ref[0] = idx + 1


def _topk_indices(scores, B, N, K):
    scores3 = scores.reshape(B, 1, N)
    idxp, idxe = pl.pallas_call(
        _topk_idx_body,
        grid=(B,),
        in_specs=[pl.BlockSpec((1, 1, N), lambda b: (b, 0, 0))],
        out_specs=[
            pl.BlockSpec((1, 1, K), lambda b: (b, 0, 0)),
            pl.BlockSpec((1, 1, K), lambda b: (b, 0, 0)),
        ],
        out_shape=[
            jax.ShapeDtypeStruct((B, 1, K), jnp.int32),
            jax.ShapeDtypeStruct((B, 1, K), jnp.int32),
        ],
    )(scores3)
    return idxp.reshape(B * K // 32, 32), idxe.reshape(B * K // 32, 32)


_NC, _NS = 2, 16
_NW = _NC * _NS
_CHUNK = 32
_NBUF = 2


def _sc_gather_body(magno_hbm, pos_hbm, idxp_hbm, idxe_hbm, out_hbm,
                    idxp_v, idxe_v, rows_v, pose_v, gsems, ssems):
    D = rows_v[0].shape[-1]
    rows_total = out_hbm.shape[0]
    rows_per_w = rows_total // _NW
    nchunk = rows_per_w // _CHUNK
    wid = lax.axis_index("s") * _NC + lax.axis_index("c")
    base = wid * rows_per_w
    cbase = wid * nchunk

    pltpu.sync_copy(idxp_hbm.at[pl.ds(cbase, nchunk)], idxp_v)
    pltpu.sync_copy(idxe_hbm.at[pl.ds(cbase, nchunk)], idxe_v)

    def start(c, slot):
        pltpu.async_copy(magno_hbm.at[idxp_v.at[c]], rows_v[slot], gsems[slot])
        pltpu.async_copy(pos_hbm.at[idxe_v.at[c]], pose_v[slot], gsems[slot])

    def finish(c, slot):
        pltpu.make_async_copy(magno_hbm.at[idxp_v.at[c]], rows_v[slot],
                              gsems[slot]).wait()
        pltpu.make_async_copy(pos_hbm.at[idxe_v.at[c]], pose_v[slot],
                              gsems[slot]).wait()

        def addrow(r, carry):
            for d0 in range(0, D, 16):
                rows_v[slot][r, pl.ds(d0, 16)] = (
                    rows_v[slot][r, pl.ds(d0, 16)]
                    + pose_v[slot][r, pl.ds(d0, 16)])
            return carry

        lax.fori_loop(0, _CHUNK, addrow, 0)
        off = pl.multiple_of(base + c * _CHUNK, _CHUNK)
        copy = pltpu.async_copy(rows_v[slot], out_hbm.at[pl.ds(off, _CHUNK)],
                                ssems[slot])
        return copy

    start(0, 0)

    def group(g, carry):
        for b in range(_NBUF):
            c = g * _NBUF + b
            nslot = (b + 1) % _NBUF

            @pl.when(c + 1 < nchunk)
            def _():
                @pl.when(c + 1 >= _NBUF)
                def _():
                    pltpu.make_async_copy(
                        rows_v[nslot],
                        out_hbm.at[pl.ds(pl.multiple_of(
                            base + (c + 1 - _NBUF) * _CHUNK, _CHUNK),
                            _CHUNK)],
                        ssems[nslot]).wait()
                start(c + 1, nslot)

            finish(c, b)
        return carry

    lax.fori_loop(0, nchunk // _NBUF, group, 0)
    for b in range(_NBUF):
        c = nchunk - _NBUF + b
        slot = c % _NBUF
        pltpu.make_async_copy(
            rows_v[slot],
            out_hbm.at[pl.ds(pl.multiple_of(base + c * _CHUNK, _CHUNK),
                             _CHUNK)],
            ssems[slot]).wait()


def _sc_gather(magno_flat, pos_flat, idxp, idxe, rows, D):
    nchunk_w = rows // _NW // _CHUNK
    mesh = plsc.VectorSubcoreMesh(core_axis_name="c", subcore_axis_name="s")
    return pl.kernel(
        _sc_gather_body,
        out_type=jax.ShapeDtypeStruct((rows, D), jnp.float32),
        mesh=mesh,
        scratch_types=[
            pltpu.VMEM((nchunk_w, _CHUNK), jnp.int32),
            pltpu.VMEM((nchunk_w, _CHUNK), jnp.int32),
            [pltpu.VMEM((_CHUNK, D), jnp.float32) for _ in range(_NBUF)],
            [pltpu.VMEM((_CHUNK, D), jnp.float32) for _ in range(_NBUF)],
            [pltpu.SemaphoreType.DMA for _ in range(_NBUF)],
            [pltpu.SemaphoreType.DMA for _ in range(_NBUF)],
        ],
    )(magno_flat, pos_flat, idxp, idxe)


def kernel(magno_patches, vit_positional_embedding, scores):
    B, N, D = magno_patches.shape
    K = N // 4
    idxp, idxe = _topk_indices(scores, B, N, K)
    magno_flat = magno_patches.reshape(B * N, D)
    pos_flat = vit_positional_embedding[0]
    out = _sc_gather(magno_flat, pos_flat, idxp, idxe, B * K, D)
    return out.reshape(B, K, D)
